# Initial kernel scaffold; baseline (speedup 1.0000x reference)
import jax
import jax.numpy as jnp
from jax.experimental import pallas as pl


def kernel(x, Wr, W1, W2, gamma, beta):
    raise NotImplementedError("write your pallas kernel here")


# speedup vs baseline: 2.1723x; 2.1723x over previous
import functools

import jax
import jax.numpy as jnp
from jax import lax
from jax.experimental import pallas as pl
from jax.experimental.pallas import tpu as pltpu

D = 1024
E = 8
TOPK = 2
H = 512
T = 2048
BT = 256


def _dense_body(x_ref, wrt_ref, w1t_ref, w2t_ref, g_ref, b_ref, o_ref):
    x = x_ref[...]
    wrt = wrt_ref[...]
    logits = jnp.dot(x, wrt, preferred_element_type=jnp.float32)
    iota_e = lax.broadcasted_iota(jnp.int32, (BT, E), 1)
    m0 = jnp.max(logits, axis=1, keepdims=True)
    is_max0 = logits >= m0
    a0 = jnp.min(jnp.where(is_max0, iota_e, E), axis=1, keepdims=True)
    onehot0 = (iota_e == a0)
    masked = jnp.where(onehot0, -jnp.inf, logits)
    m1 = jnp.max(masked, axis=1, keepdims=True)
    is_max1 = masked >= m1
    a1 = jnp.min(jnp.where(is_max1, iota_e, E), axis=1, keepdims=True)
    onehot1 = (iota_e == a1)
    d = jnp.exp(m1 - m0)
    s = 1.0 + d
    w0 = 1.0 / s
    w1 = d / s
    comb = jnp.where(onehot0, w0, 0.0) + jnp.where(onehot1, w1, 0.0)

    acc = jnp.zeros((BT, D), dtype=jnp.float32)
    for e in range(E):
        h = jnp.dot(x, w1t_ref[e], preferred_element_type=jnp.float32)
        h = h * jax.nn.sigmoid(h)
        h = jnp.dot(h, w2t_ref[e], preferred_element_type=jnp.float32)
        acc = acc + comb[:, e:e + 1] * h

    mean = jnp.mean(acc, axis=1, keepdims=True)
    cent = acc - mean
    var = jnp.mean(cent * cent, axis=1, keepdims=True)
    normed = cent * lax.rsqrt(var + 1e-5) * g_ref[...] + b_ref[...]
    o_ref[...] = normed


@jax.jit
def _dense_call(xf, wrt, w1t, w2t, gamma2, beta2):
    return pl.pallas_call(
        _dense_body,
        grid=(T // BT,),
        in_specs=[
            pl.BlockSpec((BT, D), lambda i: (i, 0)),
            pl.BlockSpec((D, E), lambda i: (0, 0)),
            pl.BlockSpec((E, D, H), lambda i: (0, 0, 0)),
            pl.BlockSpec((E, H, D), lambda i: (0, 0, 0)),
            pl.BlockSpec((1, D), lambda i: (0, 0)),
            pl.BlockSpec((1, D), lambda i: (0, 0)),
        ],
        out_specs=pl.BlockSpec((BT, D), lambda i: (i, 0)),
        out_shape=jax.ShapeDtypeStruct((T, D), jnp.float32),
    )(xf, wrt, w1t, w2t, gamma2, beta2)


def kernel(x, Wr, W1, W2, gamma, beta):
    B, S, Dm = x.shape
    xf = x.reshape(-1, Dm)
    wrt = Wr.T
    w1t = W1.transpose(0, 2, 1)
    w2t = W2.transpose(0, 2, 1)
    out = _dense_call(xf, wrt, w1t, w2t,
                      gamma.reshape(1, Dm), beta.reshape(1, Dm))
    return out.reshape(B, S, Dm)


# speedup vs baseline: 3.8109x; 1.7543x over previous
import functools

import jax
import jax.numpy as jnp
from jax import lax
from jax.experimental import pallas as pl
from jax.experimental.pallas import tpu as pltpu

D = 1024
E = 8
TOPK = 2
H = 512
T = 2048
BT = 256


def _dot_t(a, b):
    return lax.dot_general(a, b, (((1,), (1,)), ((), ())),
                           preferred_element_type=jnp.float32)


def _dense_body(x_ref, wr_ref, w1_ref, w2_ref, g_ref, b_ref, o_ref):
    x = x_ref[...]
    logits = _dot_t(x, wr_ref[...])
    iota_e = lax.broadcasted_iota(jnp.int32, (BT, E), 1)
    m0 = jnp.max(logits, axis=1, keepdims=True)
    is_max0 = logits >= m0
    a0 = jnp.min(jnp.where(is_max0, iota_e, E), axis=1, keepdims=True)
    onehot0 = (iota_e == a0)
    masked = jnp.where(onehot0, -jnp.inf, logits)
    m1 = jnp.max(masked, axis=1, keepdims=True)
    is_max1 = masked >= m1
    a1 = jnp.min(jnp.where(is_max1, iota_e, E), axis=1, keepdims=True)
    onehot1 = (iota_e == a1)
    d = jnp.exp(m1 - m0)
    s = 1.0 + d
    w0 = 1.0 / s
    w1 = d / s
    comb = jnp.where(onehot0, w0, 0.0) + jnp.where(onehot1, w1, 0.0)

    acc = jnp.zeros((BT, D), dtype=jnp.float32)
    for e in range(E):
        h = _dot_t(x, w1_ref[e])
        h = h * jax.nn.sigmoid(h)
        h = _dot_t(h, w2_ref[e])
        acc = acc + comb[:, e:e + 1] * h

    mean = jnp.mean(acc, axis=1, keepdims=True)
    cent = acc - mean
    var = jnp.mean(cent * cent, axis=1, keepdims=True)
    normed = cent * lax.rsqrt(var + 1e-5) * g_ref[...] + b_ref[...]
    o_ref[...] = normed


@jax.jit
def _dense_call(xf, wr, w1, w2, gamma2, beta2):
    return pl.pallas_call(
        _dense_body,
        grid=(T // BT,),
        in_specs=[
            pl.BlockSpec((BT, D), lambda i: (i, 0)),
            pl.BlockSpec((E, D), lambda i: (0, 0)),
            pl.BlockSpec((E, H, D), lambda i: (0, 0, 0)),
            pl.BlockSpec((E, D, H), lambda i: (0, 0, 0)),
            pl.BlockSpec((1, D), lambda i: (0, 0)),
            pl.BlockSpec((1, D), lambda i: (0, 0)),
        ],
        out_specs=pl.BlockSpec((BT, D), lambda i: (i, 0)),
        out_shape=jax.ShapeDtypeStruct((T, D), jnp.float32),
    )(xf, wr, w1, w2, gamma2, beta2)


def kernel(x, Wr, W1, W2, gamma, beta):
    B, S, Dm = x.shape
    xf = x.reshape(-1, Dm)
    out = _dense_call(xf, Wr, W1, W2,
                      gamma.reshape(1, Dm), beta.reshape(1, Dm))
    return out.reshape(B, S, Dm)
